# baseline (speedup 1.0000x reference)
import functools
import math

import jax
import jax.numpy as jnp
from jax import lax
from jax.experimental import pallas as pl
from jax.experimental.pallas import tpu as pltpu
from jax.experimental.pallas import tpu_sc as plsc

D_MODEL = 64
B, S = 4096, 200
P = B * S

NC, NS = 2, 16
NW = NC * NS
P_PER_W = P // NW

CH = 1024
G = P_PER_W // CH
SUB = 256
N_SUB = CH // SUB
NBUF = 2

_mesh = plsc.VectorSubcoreMesh(core_axis_name="c", subcore_axis_name="s")


@functools.partial(
    pl.kernel,
    mesh=_mesh,
    compiler_params=pltpu.CompilerParams(use_tc_tiling_on_sc=False),
    out_type=jax.ShapeDtypeStruct((P, D_MODEL), jnp.float32),
    scratch_types=[
        pltpu.VMEM((NBUF * CH,), jnp.int32),
        pltpu.VMEM((NBUF, SUB, D_MODEL), jnp.float32),
        [pltpu.SemaphoreType.DMA] * NBUF,
        [pltpu.SemaphoreType.DMA] * NBUF,
    ],
)
def _emb_lookup(x_hbm, table_hbm, out_hbm, idx_v, rows_v, gsems, wsems):
    wid = lax.axis_index("s") * NC + lax.axis_index("c")
    p_base = wid * P_PER_W

    def stage_idx(h, bi):
        pltpu.sync_copy(
            x_hbm.at[pl.ds(p_base + h * CH, CH)],
            idx_v.at[pl.ds(bi * CH, CH)],
        )

    def start_gather(k, bi, br):
        pltpu.async_copy(
            table_hbm.at[idx_v.at[pl.ds(bi * CH + k * SUB, SUB)]],
            rows_v.at[br],
            gsems[br],
        )

    def drain_gather(br):
        pltpu.make_async_copy(
            out_hbm.at[pl.ds(0, SUB)], rows_v.at[br], gsems[br]
        ).wait()

    def start_writeback(h, k, br):
        pltpu.async_copy(
            rows_v.at[br],
            out_hbm.at[pl.ds(p_base + h * CH + k * SUB, SUB)],
            wsems[br],
        )

    def drain_writeback(br):
        pltpu.make_async_copy(
            rows_v.at[br], out_hbm.at[pl.ds(0, SUB)], wsems[br]
        ).wait()

    def scale(br):
        def body(r, c2):
            rows_v[br, r, :] = rows_v[br, r, :] * 8.0
            return c2

        lax.fori_loop(0, SUB, body, 0, unroll=8)

    def chunk(h, bi, h_next, first, last):
        for k in range(N_SUB):
            br = k % NBUF
            nk = (k + 1) % N_SUB
            nbi = 1 - bi if nk == 0 else bi
            last_sub = last and k == N_SUB - 1

            if not last_sub:
                if not (first and k == 0):
                    drain_writeback(1 - br)
                if nk == 0 and not last:
                    stage_idx(h_next, nbi)
                start_gather(nk, nbi, 1 - br)

            drain_gather(br)
            scale(br)
            start_writeback(h, k, br)

    def pair(t, carry):
        h0 = 2 * t
        chunk(h0, 0, h0 + 1, False, False)
        chunk(h0 + 1, 1, h0 + 2, False, False)
        return carry

    stage_idx(0, 0)
    start_gather(0, 0, 0)
    chunk(0, 0, 1, True, False)
    chunk(1, 1, 2, False, False)
    lax.fori_loop(1, (G - 1) // 2, pair, 0)
    chunk(G - 1, 0, G - 1, False, True)
    drain_writeback(0)
    drain_writeback(1)


def kernel(x, table):
    xf = x.astype(jnp.int32).reshape(P)
    out = _emb_lookup(xf, table)
    return out.reshape(B, S, D_MODEL)
